# baseline (speedup 1.0000x reference)
import functools

import jax
import jax.numpy as jnp
from jax import lax
from jax.experimental import pallas as pl
from jax.experimental.pallas import tpu as pltpu
from jax.experimental.pallas import tpu_sc as plsc

_FEM_WEIGHT = 1.0
_REG_WEIGHT = 0.1


def _make_sc_kernel(n, X, YZ):
    mesh = plsc.VectorSubcoreMesh(core_axis_name="c", subcore_axis_name="s")
    NC = 2
    planes = 8
    n_workers = (n * X) // planes

    @functools.partial(
        pl.kernel,
        mesh=mesh,
        out_type=jax.ShapeDtypeStruct((32 * 16,), jnp.float32),
        scratch_types=[
            pltpu.VMEM((planes, YZ), jnp.float32),
            pltpu.VMEM((planes, YZ), jnp.float32),
            pltpu.VMEM((planes, YZ), jnp.float32),
            pltpu.VMEM((1, YZ), jnp.float32),
            pltpu.VMEM((16,), jnp.float32),
            pltpu.SemaphoreType.DMA((4,)),
        ],
    )
    def sck(nm_hbm, fm_hbm, pr_hbm, out_hbm, nm_v, fm_v, pr_v, halo_v, out_v, sems):
        wid = lax.axis_index("s") * NC + lax.axis_index("c")
        w = jnp.minimum(wid, n_workers - 1)
        half = w % 2
        s = (w // 2) * X + half * planes
        halo_row = jnp.minimum(s + planes, n * X - planes)

        c_nm = pltpu.async_copy(nm_hbm.at[pl.ds(s, planes)], nm_v, sems.at[0])
        c_fm = pltpu.async_copy(fm_hbm.at[pl.ds(s, planes)], fm_v, sems.at[1])
        c_pr = pltpu.async_copy(pr_hbm.at[pl.ds(s, planes)], pr_v, sems.at[2])
        c_ha = pltpu.async_copy(pr_hbm.at[pl.ds(halo_row, 1)], halo_v, sems.at[3])
        c_nm.wait()
        c_fm.wait()

        femv = jnp.zeros((16,), jnp.float32)
        for j in range(planes):
            for y in range(16):
                d = nm_v[j, pl.ds(y * 16, 16)] - fm_v[j, pl.ds(y * 16, 16)]
                femv = femv + d * d

        c_pr.wait()
        c_ha.wait()
        mz = jnp.where(lax.iota(jnp.int32, 16) < 15, 1.0, 0.0).astype(jnp.float32)
        w7 = jnp.where(half == 0, 1.0, 0.0).astype(jnp.float32)

        regv = jnp.zeros((16,), jnp.float32)
        for j in range(planes):
            last = j == planes - 1
            for y in range(15):
                cur = pr_v[j, pl.ds(y * 16, 16)]
                dz = (pr_v[j, pl.ds(y * 16 + 1, 16)] - cur) * mz
                dy = (pr_v[j, pl.ds((y + 1) * 16, 16)] - cur) * mz
                if last:
                    nxt = halo_v[0, pl.ds(y * 16, 16)]
                else:
                    nxt = pr_v[j + 1, pl.ds(y * 16, 16)]
                dx = (nxt - cur) * mz
                t = dz * dz + dy * dy + dx * dx
                if last:
                    t = t * w7
                regv = regv + t

        act = jnp.where(wid < n_workers, 1.0, 0.0).astype(jnp.float32)
        n_total = float(n * X * YZ)
        val = (femv * (_FEM_WEIGHT / n_total) + regv * (_REG_WEIGHT / n)) * act
        out_v[...] = val
        pltpu.sync_copy(out_v, out_hbm.at[pl.ds(wid * 16, 16)])

    return sck


def kernel(network_mesh, pc, fem_mesh, pred):
    del pc
    B, C, X, Y, Z = network_mesh.shape
    n = B * C
    nm = network_mesh.reshape(n * X, Y * Z)
    fm = fem_mesh.reshape(n * X, Y * Z)
    pr = pred.reshape(n * X, Y * Z)
    sck = _make_sc_kernel(n, X, Y * Z)
    partials = sck(nm, fm, pr)
    return jnp.sum(partials)


# speedup vs baseline: 6.4248x; 6.4248x over previous
import jax
import jax.numpy as jnp
from jax.experimental import pallas as pl
from jax.experimental.pallas import tpu as pltpu

_FEM_WEIGHT = 1.0
_REG_WEIGHT = 0.1
_FM_CHUNKS = 4


def _loss_kernel(nm_hbm, fm_hbm, pr_hbm, out_ref, nm_v, fm_v, pr_v, sems):
    n = nm_v.shape[0]
    rows = n // _FM_CHUNKS

    cp_pr = pltpu.make_async_copy(pr_hbm, pr_v, sems.at[0])
    cp_nm = pltpu.make_async_copy(nm_hbm, nm_v, sems.at[1])
    cp_pr.start()
    cp_nm.start()
    cp_fm = []
    for c in range(_FM_CHUNKS):
        sl = pl.ds(c * rows, rows)
        cp = pltpu.make_async_copy(fm_hbm.at[sl], fm_v.at[sl], sems.at[2 + c])
        cp.start()
        cp_fm.append(cp)

    cp_pr.wait()
    p = pr_v[...]
    core = p[:, :-1, :-1, :-1]
    dx = p[:, 1:, :-1, :-1] - core
    dy = p[:, :-1, 1:, :-1] - core
    dz = p[:, :-1, :-1, 1:] - core
    reg = jnp.sum(dx * dx) + jnp.sum(dy * dy) + jnp.sum(dz * dz)

    cp_nm.wait()
    fem = 0.0
    for c in range(_FM_CHUNKS):
        sl = pl.ds(c * rows, rows)
        cp_fm[c].wait()
        d = nm_v[sl] - fm_v[sl]
        fem = fem + jnp.sum(d * d)

    n_total = 1.0
    for s in nm_v.shape:
        n_total *= s
    n_bc = n
    out_ref[0, 0] = fem * (_FEM_WEIGHT / n_total) + reg * (_REG_WEIGHT / n_bc)


def kernel(network_mesh, pc, fem_mesh, pred):
    del pc
    B, C, X, Y, Z = network_mesh.shape
    n = B * C
    nm = network_mesh.reshape(n, X, Y, Z)
    fm = fem_mesh.reshape(n, X, Y, Z)
    pr = pred.reshape(n, X, Y, Z)
    any_spec = pl.BlockSpec(memory_space=pl.ANY)
    out = pl.pallas_call(
        _loss_kernel,
        out_shape=jax.ShapeDtypeStruct((1, 1), jnp.float32),
        in_specs=[any_spec, any_spec, any_spec],
        out_specs=pl.BlockSpec(memory_space=pltpu.SMEM),
        scratch_shapes=[
            pltpu.VMEM((n, X, Y, Z), jnp.float32),
            pltpu.VMEM((n, X, Y, Z), jnp.float32),
            pltpu.VMEM((n, X, Y, Z), jnp.float32),
            pltpu.SemaphoreType.DMA((2 + _FM_CHUNKS,)),
        ],
    )(nm, fm, pr)
    return out[0, 0]
